# baseline (speedup 1.0000x reference)
import functools

import jax
import jax.numpy as jnp
from jax import lax
from jax.experimental import pallas as pl
from jax.experimental.pallas import tpu as pltpu
from jax.experimental.pallas import tpu_sc as plsc

_NC = 2
_NS = 16
_NW = _NC * _NS

_LN = 128


def _build_gather(BT, S, V, D, DP):
    n_strips = DP // _LN
    last_h = D - (n_strips - 1) * _LN
    blocks = S * (BT // _LN)
    bpw = blocks // _NW
    bb = BT // _LN
    assert n_strips % 2 == 0 and last_h % 8 == 0 and blocks % _NW == 0
    mesh = plsc.VectorSubcoreMesh(core_axis_name="c", subcore_axis_name="s")

    @functools.partial(
        pl.kernel,
        mesh=mesh,
        out_type=jax.ShapeDtypeStruct((S, D, BT), jnp.float32),
        scratch_types=[
            pltpu.VMEM((bpw, _LN), jnp.int32),
            pltpu.VMEM((_LN, _LN), jnp.float32),
            pltpu.VMEM((_LN, _LN), jnp.float32),
            pltpu.VMEM((_LN, _LN), jnp.float32),
            pltpu.VMEM((_LN, _LN), jnp.float32),
            pltpu.SemaphoreType.DMA,
            pltpu.SemaphoreType.DMA,
            pltpu.SemaphoreType.DMA,
            pltpu.SemaphoreType.DMA,
        ],
        compiler_params=pltpu.CompilerParams(needs_layout_passes=False),
    )
    def gather_kernel(tabs_hbm, idx_hbm, z_hbm, idx_v,
                      raw0, raw1, tr0, tr1, gsem0, gsem1, ssem0, ssem1):
        wid = lax.axis_index("s") * _NC + lax.axis_index("c")
        bufs = ((raw0, tr0, gsem0, ssem0), (raw1, tr1, gsem1, ssem1))
        pltpu.sync_copy(idx_hbm.at[wid], idx_v)
        rows16 = lax.iota(jnp.int32, 16)

        def fire_gather(beta, k, rb, gs):
            pltpu.async_copy(tabs_hbm.at[k].at[idx_v.at[beta]], rb, gs)

        def wait_gather(rb, gs):
            pltpu.make_async_copy(
                tabs_hbm.at[0].at[idx_v.at[0]], rb, gs).wait()

        def store_dst(s, b0, k):
            if k < n_strips - 1:
                return z_hbm.at[s, pl.ds(k * _LN, _LN), pl.ds(b0, _LN)]
            return z_hbm.at[s, pl.ds(k * _LN, last_h), pl.ds(b0, _LN)]

        def fire_store(s, b0, k, tb, ss):
            if k < n_strips - 1:
                pltpu.async_copy(tb, store_dst(s, b0, k), ss)
            else:
                pltpu.async_copy(tb.at[pl.ds(0, last_h)],
                                 store_dst(s, b0, k), ss)

        def wait_store(k, tb, ss):
            if k < n_strips - 1:
                pltpu.make_async_copy(tb, store_dst(0, 0, 0), ss).wait()
            else:
                pltpu.make_async_copy(tb.at[pl.ds(0, last_h)],
                                      store_dst(0, 0, k), ss).wait()

        def transpose(rb, tb):
            def cc_body(cc, carry):
                for t in range(8):
                    cp = cc * 8 + t
                    col = jnp.full((16,), 0, jnp.int32) + cp
                    for m in range(8):
                        v = plsc.load_gather(rb, [rows16 + 16 * m, col])
                        tb[cp, pl.ds(16 * m, 16)] = v
                return carry
            lax.fori_loop(0, _LN // 8, cc_body, 0)

        fire_gather(0, 0, raw0, gsem0)

        def block_body(beta, carry):
            gb = wid * bpw + beta
            s = gb // bb
            b0 = (gb % bb) * _LN
            for k in range(n_strips):
                rb, tb, gs, ss = bufs[k % 2]
                nrb, _, ngs, _ = bufs[(k + 1) % 2]
                if k < n_strips - 1:
                    fire_gather(beta, k + 1, nrb, ngs)
                else:
                    @pl.when(beta + 1 < bpw)
                    def _():
                        fire_gather(beta + 1, 0, nrb, ngs)
                wait_gather(rb, gs)
                prev_k = k - 2 if k >= 2 else k + n_strips - 2
                if k >= 2:
                    wait_store(prev_k, tb, ss)
                else:
                    @pl.when(beta > 0)
                    def _():
                        wait_store(prev_k, tb, ss)
                transpose(rb, tb)
                fire_store(s, b0, k, tb, ss)
            return carry

        lax.fori_loop(0, bpw, block_body, 0)
        for k in (n_strips - 2, n_strips - 1):
            _, tb, _, ss = bufs[k % 2]
            wait_store(k, tb, ss)

    return gather_kernel


def kernel(x, table):
    BT, S = x.shape
    V, D = table.shape
    DP = (D + _LN - 1) // _LN * _LN
    tablep = jnp.pad(table, ((0, 0), (0, DP - D)))
    tabs = tablep.reshape(V, DP // _LN, _LN).transpose(1, 0, 2)
    xb = x.T.reshape(S * (BT // _LN), _LN).reshape(_NW, -1, _LN)
    xb = xb.astype(jnp.int32)
    z = _build_gather(BT, S, V, D, DP)(tabs, xb)
    return jnp.transpose(z, (2, 0, 1))


# speedup vs baseline: 1.8973x; 1.8973x over previous
import functools

import jax
import jax.numpy as jnp
from jax import lax
from jax.experimental import pallas as pl
from jax.experimental.pallas import tpu as pltpu
from jax.experimental.pallas import tpu_sc as plsc

_NC = 2
_NS = 16
_NW = _NC * _NS

_LN = 128


def _build_gather(BT, S, V, D, DP):
    n_strips = DP // _LN
    last_h = D - (n_strips - 1) * _LN
    blocks = S * (BT // _LN)
    bpw = blocks // _NW
    bb = BT // _LN
    assert n_strips % 2 == 0 and last_h % 8 == 0 and blocks % _NW == 0
    mesh = plsc.VectorSubcoreMesh(core_axis_name="c", subcore_axis_name="s")

    @functools.partial(
        pl.kernel,
        mesh=mesh,
        out_type=jax.ShapeDtypeStruct((S, D, BT), jnp.float32),
        scratch_types=[
            pltpu.VMEM((bpw, _LN), jnp.int32),
            pltpu.VMEM((_LN, _LN), jnp.float32),
            pltpu.VMEM((_LN, _LN), jnp.float32),
            pltpu.VMEM((_LN, _LN), jnp.float32),
            pltpu.VMEM((_LN, _LN), jnp.float32),
            pltpu.SemaphoreType.DMA,
            pltpu.SemaphoreType.DMA,
            pltpu.SemaphoreType.DMA,
            pltpu.SemaphoreType.DMA,
        ],
        compiler_params=pltpu.CompilerParams(needs_layout_passes=False),
    )
    def gather_kernel(tabs_hbm, idx_hbm, z_hbm, idx_v,
                      raw0, raw1, tr0, tr1, gsem0, gsem1, ssem0, ssem1):
        wid = lax.axis_index("s") * _NC + lax.axis_index("c")
        bufs = ((raw0, tr0, gsem0, ssem0), (raw1, tr1, gsem1, ssem1))
        pltpu.sync_copy(idx_hbm.at[wid], idx_v)
        rows16 = lax.iota(jnp.int32, 16)

        def fire_gather(beta, k, rb, gs):
            pltpu.async_copy(tabs_hbm.at[k].at[idx_v.at[beta]], rb, gs)

        def wait_gather(rb, gs):
            pltpu.make_async_copy(
                tabs_hbm.at[0].at[idx_v.at[0]], rb, gs).wait()

        def store_dst(s, b0, k):
            if k < n_strips - 1:
                return z_hbm.at[s, pl.ds(k * _LN, _LN), pl.ds(b0, _LN)]
            return z_hbm.at[s, pl.ds(k * _LN, last_h), pl.ds(b0, _LN)]

        def fire_store(s, b0, k, tb, ss):
            if k < n_strips - 1:
                pltpu.async_copy(tb, store_dst(s, b0, k), ss)
            else:
                pltpu.async_copy(tb.at[pl.ds(0, last_h)],
                                 store_dst(s, b0, k), ss)

        def wait_store(k, tb, ss):
            if k < n_strips - 1:
                pltpu.make_async_copy(tb, store_dst(0, 0, 0), ss).wait()
            else:
                pltpu.make_async_copy(tb.at[pl.ds(0, last_h)],
                                      store_dst(0, 0, k), ss).wait()

        row_ids = [rows16 + 16 * m for m in range(8)]

        def transpose(rb, tb):
            @plsc.parallel_loop(0, _LN, unroll=4)
            def _(cp):
                col = jnp.full((16,), 0, jnp.int32) + cp
                for m in range(8):
                    v = plsc.load_gather(rb, [row_ids[m], col])
                    tb[cp, pl.ds(16 * m, 16)] = v

        fire_gather(0, 0, raw0, gsem0)

        def block_body(beta, carry):
            gb = wid * bpw + beta
            s = gb // bb
            b0 = (gb % bb) * _LN
            for k in range(n_strips):
                rb, tb, gs, ss = bufs[k % 2]
                nrb, _, ngs, _ = bufs[(k + 1) % 2]
                if k < n_strips - 1:
                    fire_gather(beta, k + 1, nrb, ngs)
                else:
                    @pl.when(beta + 1 < bpw)
                    def _():
                        fire_gather(beta + 1, 0, nrb, ngs)
                wait_gather(rb, gs)
                prev_k = k - 2 if k >= 2 else k + n_strips - 2
                if k >= 2:
                    wait_store(prev_k, tb, ss)
                else:
                    @pl.when(beta > 0)
                    def _():
                        wait_store(prev_k, tb, ss)
                transpose(rb, tb)
                fire_store(s, b0, k, tb, ss)
            return carry

        lax.fori_loop(0, bpw, block_body, 0)
        for k in (n_strips - 2, n_strips - 1):
            _, tb, _, ss = bufs[k % 2]
            wait_store(k, tb, ss)

    return gather_kernel


def kernel(x, table):
    BT, S = x.shape
    V, D = table.shape
    DP = (D + _LN - 1) // _LN * _LN
    tablep = jnp.pad(table, ((0, 0), (0, DP - D)))
    tabs = tablep.reshape(V, DP // _LN, _LN).transpose(1, 0, 2)
    xb = x.T.reshape(S * (BT // _LN), _LN).reshape(_NW, -1, _LN)
    xb = xb.astype(jnp.int32)
    z = _build_gather(BT, S, V, D, DP)(tabs, xb)
    return jnp.transpose(z, (2, 0, 1))


# speedup vs baseline: 1.8990x; 1.0009x over previous
import functools

import jax
import jax.numpy as jnp
from jax import lax
from jax.experimental import pallas as pl
from jax.experimental.pallas import tpu as pltpu
from jax.experimental.pallas import tpu_sc as plsc

_NC = 2
_NS = 16
_NW = _NC * _NS

_LN = 128


def _build_gather(BT, S, V, D, DP):
    n_strips = DP // _LN
    last_h = D - (n_strips - 1) * _LN
    blocks = S * (BT // _LN)
    bpw = blocks // _NW
    bb = BT // _LN
    assert n_strips % 2 == 0 and last_h % 8 == 0 and blocks % _NW == 0
    mesh = plsc.VectorSubcoreMesh(core_axis_name="c", subcore_axis_name="s")

    @functools.partial(
        pl.kernel,
        mesh=mesh,
        out_type=jax.ShapeDtypeStruct((S, D, BT), jnp.float32),
        scratch_types=[
            pltpu.VMEM((bpw, _LN), jnp.int32),
            pltpu.VMEM((_LN, _LN), jnp.float32),
            pltpu.VMEM((_LN, _LN), jnp.float32),
            pltpu.VMEM((_LN, _LN), jnp.float32),
            pltpu.VMEM((_LN, _LN), jnp.float32),
            pltpu.SemaphoreType.DMA,
            pltpu.SemaphoreType.DMA,
            pltpu.SemaphoreType.DMA,
            pltpu.SemaphoreType.DMA,
        ],
        compiler_params=pltpu.CompilerParams(needs_layout_passes=False),
    )
    def gather_kernel(tabs_hbm, idx_hbm, z_hbm, idx_v,
                      raw0, raw1, tr0, tr1, gsem0, gsem1, ssem0, ssem1):
        wid = lax.axis_index("s") * _NC + lax.axis_index("c")
        bufs = ((raw0, tr0, gsem0, ssem0), (raw1, tr1, gsem1, ssem1))
        pltpu.sync_copy(idx_hbm.at[wid], idx_v)
        rows16 = lax.iota(jnp.int32, 16)

        def fire_gather(beta, k, rb, gs):
            pltpu.async_copy(tabs_hbm.at[k].at[idx_v.at[beta]], rb, gs)

        def wait_gather(rb, gs):
            pltpu.make_async_copy(
                tabs_hbm.at[0].at[idx_v.at[0]], rb, gs).wait()

        def store_dst(s, b0, k):
            if k < n_strips - 1:
                return z_hbm.at[s, pl.ds(k * _LN, _LN), pl.ds(b0, _LN)]
            return z_hbm.at[s, pl.ds(k * _LN, last_h), pl.ds(b0, _LN)]

        def fire_store(s, b0, k, tb, ss):
            if k < n_strips - 1:
                pltpu.async_copy(tb, store_dst(s, b0, k), ss)
            else:
                pltpu.async_copy(tb.at[pl.ds(0, last_h)],
                                 store_dst(s, b0, k), ss)

        def wait_store(k, tb, ss):
            if k < n_strips - 1:
                pltpu.make_async_copy(tb, store_dst(0, 0, 0), ss).wait()
            else:
                pltpu.make_async_copy(tb.at[pl.ds(0, last_h)],
                                      store_dst(0, 0, k), ss).wait()

        row_ids = [rows16 + 16 * m for m in range(8)]

        def transpose(rb, tb):
            @plsc.parallel_loop(0, _LN, unroll=8)
            def _(cp):
                col = jnp.full((16,), 0, jnp.int32) + cp
                for m in range(8):
                    v = plsc.load_gather(rb, [row_ids[m], col])
                    tb[cp, pl.ds(16 * m, 16)] = v

        fire_gather(0, 0, raw0, gsem0)

        def block_body(beta, carry):
            gb = wid * bpw + beta
            s = gb // bb
            b0 = (gb % bb) * _LN
            for k in range(n_strips):
                rb, tb, gs, ss = bufs[k % 2]
                nrb, _, ngs, _ = bufs[(k + 1) % 2]
                if k < n_strips - 1:
                    fire_gather(beta, k + 1, nrb, ngs)
                else:
                    @pl.when(beta + 1 < bpw)
                    def _():
                        fire_gather(beta + 1, 0, nrb, ngs)
                wait_gather(rb, gs)
                prev_k = k - 2 if k >= 2 else k + n_strips - 2
                if k >= 2:
                    wait_store(prev_k, tb, ss)
                else:
                    @pl.when(beta > 0)
                    def _():
                        wait_store(prev_k, tb, ss)
                transpose(rb, tb)
                fire_store(s, b0, k, tb, ss)
            return carry

        lax.fori_loop(0, bpw, block_body, 0)
        for k in (n_strips - 2, n_strips - 1):
            _, tb, _, ss = bufs[k % 2]
            wait_store(k, tb, ss)

    return gather_kernel


def kernel(x, table):
    BT, S = x.shape
    V, D = table.shape
    DP = (D + _LN - 1) // _LN * _LN
    tablep = jnp.pad(table, ((0, 0), (0, DP - D)))
    tabs = tablep.reshape(V, DP // _LN, _LN).transpose(1, 0, 2)
    xb = x.T.reshape(S * (BT // _LN), _LN).reshape(_NW, -1, _LN)
    xb = xb.astype(jnp.int32)
    z = _build_gather(BT, S, V, D, DP)(tabs, xb)
    return jnp.transpose(z, (2, 0, 1))


# speedup vs baseline: 7.3122x; 3.8506x over previous
import functools

import jax
import jax.numpy as jnp
from jax import lax
from jax.experimental import pallas as pl
from jax.experimental.pallas import tpu as pltpu
from jax.experimental.pallas import tpu_sc as plsc

_NC = 2
_NS = 16
_NW = _NC * _NS

_LN = 128


def _build_gather(BT, S, V, D, DP):
    n_strips = DP // _LN
    last_h = D - (n_strips - 1) * _LN
    blocks = S * (BT // _LN)
    bpw = blocks // _NW
    bb = BT // _LN
    assert n_strips % 2 == 0 and last_h % 8 == 0 and blocks % _NW == 0
    mesh = plsc.VectorSubcoreMesh(core_axis_name="c", subcore_axis_name="s")

    @functools.partial(
        pl.kernel,
        mesh=mesh,
        out_type=jax.ShapeDtypeStruct((S, D, BT), jnp.float32),
        scratch_types=[
            pltpu.VMEM((bpw, _LN), jnp.int32),
            pltpu.VMEM((_LN, _LN), jnp.float32),
            pltpu.VMEM((_LN, _LN), jnp.float32),
            pltpu.VMEM((_LN, _LN), jnp.float32),
            pltpu.VMEM((_LN, _LN), jnp.float32),
            pltpu.SemaphoreType.DMA,
            pltpu.SemaphoreType.DMA,
            pltpu.SemaphoreType.DMA,
            pltpu.SemaphoreType.DMA,
        ],
        compiler_params=pltpu.CompilerParams(needs_layout_passes=False),
    )
    def gather_kernel(tabs_hbm, idx_hbm, z_hbm, idx_v,
                      raw0, raw1, tr0, tr1, gsem0, gsem1, ssem0, ssem1):
        wid = lax.axis_index("s") * _NC + lax.axis_index("c")
        bufs = ((raw0, tr0, gsem0, ssem0), (raw1, tr1, gsem1, ssem1))
        pltpu.sync_copy(idx_hbm.at[wid], idx_v)
        rows16 = lax.iota(jnp.int32, 16)

        def fire_gather(beta, k, rb, gs):
            pltpu.async_copy(tabs_hbm.at[k].at[idx_v.at[beta]], rb, gs)

        def wait_gather(rb, gs):
            pltpu.make_async_copy(
                tabs_hbm.at[0].at[idx_v.at[0]], rb, gs).wait()

        def store_dst(s, b0, k):
            if k < n_strips - 1:
                return z_hbm.at[s, pl.ds(k * _LN, _LN), pl.ds(b0, _LN)]
            return z_hbm.at[s, pl.ds(k * _LN, last_h), pl.ds(b0, _LN)]

        def fire_store(s, b0, k, tb, ss):
            if k < n_strips - 1:
                pltpu.async_copy(tb, store_dst(s, b0, k), ss)
            else:
                pltpu.async_copy(tb.at[pl.ds(0, last_h)],
                                 store_dst(s, b0, k), ss)

        def wait_store(k, tb, ss):
            if k < n_strips - 1:
                pltpu.make_async_copy(tb, store_dst(0, 0, 0), ss).wait()
            else:
                pltpu.make_async_copy(tb.at[pl.ds(0, last_h)],
                                      store_dst(0, 0, k), ss).wait()

        row_ids = [rows16 + 16 * m for m in range(8)]

        def transpose(rb, tb):
            @plsc.parallel_loop(0, _LN, unroll=4)
            def _(cp):
                for m in range(8):
                    rr = row_ids[m]
                    cc = (cp + rr) & (_LN - 1)
                    v = plsc.load_gather(rb, [rr, cc])
                    plsc.store_scatter(tb, [cc, rr], v)

        fire_gather(0, 0, raw0, gsem0)

        def block_body(beta, carry):
            gb = wid * bpw + beta
            s = gb // bb
            b0 = (gb % bb) * _LN
            for k in range(n_strips):
                rb, tb, gs, ss = bufs[k % 2]
                nrb, _, ngs, _ = bufs[(k + 1) % 2]
                if k < n_strips - 1:
                    fire_gather(beta, k + 1, nrb, ngs)
                else:
                    @pl.when(beta + 1 < bpw)
                    def _():
                        fire_gather(beta + 1, 0, nrb, ngs)
                wait_gather(rb, gs)
                prev_k = k - 2 if k >= 2 else k + n_strips - 2
                if k >= 2:
                    wait_store(prev_k, tb, ss)
                else:
                    @pl.when(beta > 0)
                    def _():
                        wait_store(prev_k, tb, ss)
                transpose(rb, tb)
                fire_store(s, b0, k, tb, ss)
            return carry

        lax.fori_loop(0, bpw, block_body, 0)
        for k in (n_strips - 2, n_strips - 1):
            _, tb, _, ss = bufs[k % 2]
            wait_store(k, tb, ss)

    return gather_kernel


def kernel(x, table):
    BT, S = x.shape
    V, D = table.shape
    DP = (D + _LN - 1) // _LN * _LN
    tablep = jnp.pad(table, ((0, 0), (0, DP - D)))
    tabs = tablep.reshape(V, DP // _LN, _LN).transpose(1, 0, 2)
    xb = x.T.reshape(S * (BT // _LN), _LN).reshape(_NW, -1, _LN)
    xb = xb.astype(jnp.int32)
    z = _build_gather(BT, S, V, D, DP)(tabs, xb)
    return jnp.transpose(z, (2, 0, 1))


# speedup vs baseline: 11.0673x; 1.5135x over previous
import functools

import jax
import jax.numpy as jnp
from jax import lax
from jax.experimental import pallas as pl
from jax.experimental.pallas import tpu as pltpu
from jax.experimental.pallas import tpu_sc as plsc

_NC = 2
_NS = 16
_NW = _NC * _NS

_LN = 128


def _build_gather(BT, S, V, D, DP):
    n_strips = DP // _LN
    last_h = D - (n_strips - 1) * _LN
    blocks = S * (BT // _LN)
    bpw = blocks // _NW
    bb = BT // _LN
    assert n_strips % 2 == 0 and last_h % 8 == 0 and blocks % _NW == 0
    mesh = plsc.VectorSubcoreMesh(core_axis_name="c", subcore_axis_name="s")

    @functools.partial(
        pl.kernel,
        mesh=mesh,
        out_type=jax.ShapeDtypeStruct((S, D, BT), jnp.float32),
        scratch_types=[
            pltpu.VMEM((3, _LN), jnp.int32),
            pltpu.VMEM_SHARED((DP // _LN, V, _LN), jnp.float32),
            pltpu.VMEM((_LN, _LN), jnp.float32),
            pltpu.VMEM((_LN, _LN), jnp.float32),
            pltpu.VMEM((_LN, _LN), jnp.float32),
            pltpu.VMEM((_LN, _LN), jnp.float32),
            pltpu.SemaphoreType.DMA,
            pltpu.SemaphoreType.DMA,
            pltpu.SemaphoreType.DMA,
            pltpu.SemaphoreType.DMA,
            pltpu.SemaphoreType.DMA,
        ],
        compiler_params=pltpu.CompilerParams(needs_layout_passes=False),
    )
    def gather_kernel(tabs_hbm, idx_hbm, z_hbm, idx_v, spt,
                      raw0, raw1, tr0, tr1,
                      gsem0, gsem1, ssem0, ssem1, isem):
        wid = lax.axis_index("s") * _NC + lax.axis_index("c")
        sid = lax.axis_index("s")
        bufs = ((raw0, tr0, gsem0, ssem0), (raw1, tr1, gsem1, ssem1))
        rows16 = lax.iota(jnp.int32, 16)

        rows_maj = V // _NS // 8 * 8
        tail = V - 15 * rows_maj
        for k in range(n_strips):
            @pl.when(sid < 15)
            def _():
                pltpu.sync_copy(
                    tabs_hbm.at[k].at[pl.ds(sid * rows_maj, rows_maj)],
                    spt.at[k].at[pl.ds(sid * rows_maj, rows_maj)])

            @pl.when(sid == 15)
            def _():
                pltpu.sync_copy(
                    tabs_hbm.at[k].at[pl.ds(15 * rows_maj, tail)],
                    spt.at[k].at[pl.ds(15 * rows_maj, tail)])
        plsc.subcore_barrier()

        def fire_idx(beta):
            pltpu.async_copy(idx_hbm.at[wid].at[beta], idx_v.at[beta % 3],
                             isem)

        def wait_idx(beta):
            pltpu.make_async_copy(idx_hbm.at[wid].at[0], idx_v.at[0],
                                  isem).wait()

        def fire_gather(beta, k, rb, gs):
            pltpu.async_copy(spt.at[k].at[idx_v.at[beta % 3]], rb, gs)

        def wait_gather(rb, gs):
            pltpu.make_async_copy(
                spt.at[0].at[idx_v.at[0]], rb, gs).wait()

        def store_dst(s, b0, k):
            if k < n_strips - 1:
                return z_hbm.at[s, pl.ds(k * _LN, _LN), pl.ds(b0, _LN)]
            return z_hbm.at[s, pl.ds(k * _LN, last_h), pl.ds(b0, _LN)]

        def fire_store(s, b0, k, tb, ss):
            if k < n_strips - 1:
                pltpu.async_copy(tb, store_dst(s, b0, k), ss)
            else:
                pltpu.async_copy(tb.at[pl.ds(0, last_h)],
                                 store_dst(s, b0, k), ss)

        def wait_store(k, tb, ss):
            if k < n_strips - 1:
                pltpu.make_async_copy(tb, store_dst(0, 0, 0), ss).wait()
            else:
                pltpu.make_async_copy(tb.at[pl.ds(0, last_h)],
                                      store_dst(0, 0, k), ss).wait()

        row_ids = [rows16 + 16 * m for m in range(8)]

        def transpose(rb, tb):
            @plsc.parallel_loop(0, _LN, unroll=4)
            def _(cp):
                for m in range(8):
                    rr = row_ids[m]
                    cc = (cp + rr) & (_LN - 1)
                    v = plsc.load_gather(rb, [rr, cc])
                    plsc.store_scatter(tb, [cc, rr], v)

        pltpu.sync_copy(idx_hbm.at[wid].at[0], idx_v.at[0])
        fire_idx(1)
        fire_gather(0, 0, raw0, gsem0)

        def block_body(beta, carry):
            gb = wid * bpw + beta
            s = gb // bb
            b0 = (gb % bb) * _LN
            for k in range(n_strips):
                rb, tb, gs, ss = bufs[k % 2]
                nrb, _, ngs, _ = bufs[(k + 1) % 2]
                if k < n_strips - 1:
                    fire_gather(beta, k + 1, nrb, ngs)
                else:
                    @pl.when(beta + 1 < bpw)
                    def _():
                        wait_idx(beta + 1)
                        fire_gather(beta + 1, 0, nrb, ngs)

                    @pl.when(beta + 2 < bpw)
                    def _():
                        fire_idx(beta + 2)
                wait_gather(rb, gs)
                prev_k = k - 2 if k >= 2 else k + n_strips - 2
                if k >= 2:
                    wait_store(prev_k, tb, ss)
                else:
                    @pl.when(beta > 0)
                    def _():
                        wait_store(prev_k, tb, ss)
                transpose(rb, tb)
                fire_store(s, b0, k, tb, ss)
            return carry

        lax.fori_loop(0, bpw, block_body, 0)
        for k in (n_strips - 2, n_strips - 1):
            _, tb, _, ss = bufs[k % 2]
            wait_store(k, tb, ss)

    return gather_kernel


def kernel(x, table):
    BT, S = x.shape
    V, D = table.shape
    DP = (D + _LN - 1) // _LN * _LN
    tablep = jnp.pad(table, ((0, 0), (0, DP - D)))
    tabs = tablep.reshape(V, DP // _LN, _LN).transpose(1, 0, 2)
    xb = x.T.reshape(S * (BT // _LN), _LN).reshape(_NW, -1, _LN)
    xb = xb.astype(jnp.int32)
    z = _build_gather(BT, S, V, D, DP)(tabs, xb)
    return jnp.transpose(z, (2, 0, 1))
